# baseline (speedup 1.0000x reference)
import functools

import jax
import jax.numpy as jnp
from jax.experimental import pallas as pl

N = 10000
E = 160000
KS = 5


def _spline_basis(pseudo, ks):
    _, D = pseudo.shape
    v = pseudo * (ks - 1)
    fl = jnp.floor(v)
    fr = v - fl
    fli = fl.astype(jnp.int32)
    S = 2 ** D
    bits = ((jnp.arange(S)[:, None] >> jnp.arange(D)[None, :]) & 1).astype(jnp.int32)
    b = jnp.where(bits[None, :, :] == 1, fr[:, None, :], 1.0 - fr[:, None, :])
    basis = jnp.prod(b, axis=-1)
    idx = jnp.clip(fli[:, None, :] + bits[None, :, :], 0, ks - 1)
    strides = (ks ** jnp.arange(D)).astype(jnp.int32)
    wi = jnp.sum(idx * strides[None, None, :], axis=-1)
    return basis, wi


def _elu(v):
    return jnp.where(v > 0, v, jnp.exp(jnp.minimum(v, 0.0)) - 1.0)


def _mlp_body(h_ref, w1_ref, b1_ref, w2_ref, b2_ref, o_ref):
    h = h_ref[...]
    t = _elu(h @ w1_ref[...] + b1_ref[...][None, :])
    o_ref[...] = t @ w2_ref[...] + b2_ref[...][None, :]


def _mlp(h, w1, b1, w2, b2):
    BM = 1000
    grid = (N // BM,)
    return pl.pallas_call(
        _mlp_body,
        grid=grid,
        in_specs=[
            pl.BlockSpec((BM, h.shape[1]), lambda i: (i, 0)),
            pl.BlockSpec(w1.shape, lambda i: (0, 0)),
            pl.BlockSpec(b1.shape, lambda i: (0,)),
            pl.BlockSpec(w2.shape, lambda i: (0, 0)),
            pl.BlockSpec(b2.shape, lambda i: (0,)),
        ],
        out_specs=pl.BlockSpec((BM, w2.shape[1]), lambda i: (i, 0)),
        out_shape=jax.ShapeDtypeStruct((N, w2.shape[1]), jnp.float32),
    )(h, w1, b1, w2, b2)


def _spline_conv(x, src, dst, basis, wi, W, root, bias):
    K, Cin, Cout = W.shape
    xw = (x @ W.transpose(1, 0, 2).reshape(Cin, K * Cout)).reshape(-1, K, Cout)
    msg = xw[src[:, None], wi]
    msg = jnp.sum(msg * basis[:, :, None], axis=1)
    agg = jax.ops.segment_sum(msg, dst, num_segments=N)
    return agg + x @ root + bias


def kernel(x, edge_index, edge_attr, W1, root1, b1, W2, root2, b2, W3, root3, b3, W4, root4, b4, W5, root5, b5, W6, root6, b6, lin1_w, lin1_b, lin2_w, lin2_b):
    src, dst = edge_index[0], edge_index[1]
    basis, wi = _spline_basis(edge_attr, KS)
    h = x
    for W, r, b in [(W1, root1, b1), (W2, root2, b2), (W3, root3, b3),
                    (W4, root4, b4), (W5, root5, b5), (W6, root6, b6)]:
        h = jax.nn.elu(_spline_conv(h, src, dst, basis, wi, W, r, b))
    return _mlp(h, lin1_w, lin1_b, lin2_w, lin2_b)


# speedup vs baseline: 2.0068x; 2.0068x over previous
import functools

import jax
import jax.numpy as jnp
from jax import lax
from jax.experimental import pallas as pl
from jax.experimental.pallas import tpu as pltpu
from jax.experimental.pallas import tpu_sc as plsc

N = 10000
E = 160000
KS = 5
KTOT = KS ** 4
KP = 640
S = 16
NC = 2
NSUB = 16
NW = NC * NSUB
EPT = E // NW
B = 8
NB = EPT // B
NZT = 10
NPT = N // NZT
ZCH = 40


def _elu(v):
    return jnp.where(v > 0, v, jnp.exp(jnp.minimum(v, 0.0)) - 1.0)


def _prep_body(ea_ref, src_ref, basis_ref, g8_ref, o8_ref, g4_ref, o4_ref):
    ea = ea_ref[...]
    v = ea * (KS - 1.0)
    fl = jnp.floor(v)
    fr = v - fl
    fli = fl.astype(jnp.int32)
    src = src_ref[0, 0, :]
    w = None
    k = None
    for d in range(4):
        siota = lax.broadcasted_iota(jnp.int32, (1, S), 1)
        bits_d = (siota >> d) & 1
        bitf_d = bits_d.astype(jnp.float32)
        frd = fr[:, d:d + 1]
        wd = bitf_d * frd + (1.0 - bitf_d) * (1.0 - frd)
        idxd = jnp.clip(fli[:, d:d + 1] + bits_d, 0, KS - 1)
        w = wd if w is None else w * wd
        k = idxd * (KS ** d) if k is None else k + idxd * (KS ** d)
    srcb = src[:, None]
    basis_ref[...] = w
    g8_ref[...] = (k // 8) * N + srcb
    o8_ref[...] = (k % 8) * 16
    g4_ref[...] = (k // 4) * N + srcb
    o4_ref[...] = (k % 4) * 32


def _prep(edge_attr, src):
    BE = 4000
    grid = (E // BE,)
    src3 = src.reshape(E // BE, 1, BE)
    espec = pl.BlockSpec((BE, S), lambda i: (i, 0))
    return pl.pallas_call(
        _prep_body,
        grid=grid,
        in_specs=[
            pl.BlockSpec((BE, 4), lambda i: (i, 0)),
            pl.BlockSpec((1, 1, BE), lambda i: (i, 0, 0)),
        ],
        out_specs=[espec] * 5,
        out_shape=[
            jax.ShapeDtypeStruct((E, S), jnp.float32),
            jax.ShapeDtypeStruct((E, S), jnp.int32),
            jax.ShapeDtypeStruct((E, S), jnp.int32),
            jax.ShapeDtypeStruct((E, S), jnp.int32),
            jax.ShapeDtypeStruct((E, S), jnp.int32),
        ],
    )(edge_attr, src3)


def _xw_body(gb, x_ref, w_ref, o_ref):
    acc = jnp.dot(x_ref[...], w_ref[...], preferred_element_type=jnp.float32)
    for g in range(gb):
        o_ref[g] = acc[:, g * 128:(g + 1) * 128]


def _xw_matmul(x, wflat):
    M, Cin = x.shape
    KC = wflat.shape[1]
    GP = KC // 128
    BM, GB = 1000, 16
    grid = (M // BM, GP // GB)
    return pl.pallas_call(
        functools.partial(_xw_body, GB),
        grid=grid,
        in_specs=[
            pl.BlockSpec((BM, Cin), lambda i, j: (i, 0)),
            pl.BlockSpec((Cin, GB * 128), lambda i, j: (0, j)),
        ],
        out_specs=pl.BlockSpec((GB, BM, 128), lambda i, j: (j, i, 0)),
        out_shape=jax.ShapeDtypeStruct((GP, M, 128), jnp.float32),
    )(x, wflat)


@functools.cache
def _make_sc_agg(D):
    H = D // 16
    mesh = plsc.VectorSubcoreMesh(core_axis_name="c", subcore_axis_name="s")

    @functools.partial(
        pl.kernel,
        out_type=jax.ShapeDtypeStruct((NC, N, 128), jnp.float32),
        mesh=mesh,
        scratch_types=[
            pltpu.VMEM((ZCH, 128), jnp.float32),
            pltpu.VMEM_SHARED((N, 128), jnp.float32),
        ],
    )
    def sc_agg(xw_hbm, gidx_hbm, goff_hbm, basis_hbm, dst_hbm, out_hbm,
               zb_v, agg_sh):
        cid = lax.axis_index("c")
        sid = lax.axis_index("s")
        wid = sid * NC + cid

        zrow = jnp.zeros((16,), jnp.float32)

        def zbody(i, carry):
            for h in range(8):
                zb_v[i, h * 16:(h + 1) * 16] = zrow
            return carry

        lax.fori_loop(0, ZCH, zbody, 0)

        @pl.when(sid < NZT)
        def _():
            for c in range(NPT // ZCH):
                pltpu.sync_copy(zb_v, agg_sh.at[pl.ds(sid * NPT + c * ZCH, ZCH)])

        plsc.subcore_barrier()

        ebase0 = wid * EPT

        @pl.loop(0, NB, unroll=1)
        def batch(ib):
            ebase = ebase0 + ib * B

            def inner(idx_v, off_v, bas_v, dst_v, rows_v, msg_v, gsem):
                pltpu.sync_copy(gidx_hbm.at[pl.ds(ebase * S, B * S)], idx_v)
                pltpu.sync_copy(goff_hbm.at[pl.ds(ebase * S, B * S)], off_v)
                pltpu.sync_copy(basis_hbm.at[pl.ds(ebase * S, B * S)], bas_v)
                pltpu.sync_copy(dst_hbm.at[pl.ds(ebase, B)], dst_v)
                pltpu.async_copy(xw_hbm.at[idx_v], rows_v, gsem).wait()

                @pl.loop(0, B, unroll=1)
                def ebody(j):
                    r0 = j * S
                    bvec = bas_v[pl.ds(r0, S)]
                    ovec = off_v[pl.ds(r0, S)]
                    acc = [jnp.zeros((16,), jnp.float32) for _ in range(H)]
                    for s in range(S):
                        b = bvec[s]
                        off = ovec[s]
                        for h in range(H):
                            acc[h] = acc[h] + rows_v[r0 + s, pl.ds(off + h * 16, 16)] * b
                    for h in range(H):
                        msg_v[j, h * 16:(h + 1) * 16] = acc[h]

                pltpu.sync_copy(msg_v, agg_sh.at[dst_v], add=True)

            pl.run_scoped(
                inner,
                pltpu.VMEM((B * S,), jnp.int32),
                pltpu.VMEM((B * S,), jnp.int32),
                pltpu.VMEM((B * S,), jnp.float32),
                pltpu.VMEM((B,), jnp.int32),
                pltpu.VMEM((B * S, 128), jnp.float32),
                pltpu.VMEM((B, 128), jnp.float32),
                pltpu.SemaphoreType.DMA,
            )

        plsc.subcore_barrier()

        @pl.when(sid < NZT)
        def _():
            for c in range(NPT // ZCH):
                pltpu.sync_copy(agg_sh.at[pl.ds(sid * NPT + c * ZCH, ZCH)], zb_v)
                pltpu.sync_copy(zb_v, out_hbm.at[cid, pl.ds(sid * NPT + c * ZCH, ZCH)])

    return sc_agg


def _root_body(cout, agg_ref, x_ref, r_ref, b_ref, o_ref):
    a = agg_ref[0, :, :cout] + agg_ref[1, :, :cout]
    v = a + jnp.dot(x_ref[...], r_ref[...],
                    preferred_element_type=jnp.float32) + b_ref[...][None, :]
    o_ref[...] = _elu(v)


def _root_combine(agg2, x, root, bias):
    Cin, Cout = root.shape
    BM = 2000
    grid = (N // BM,)
    return pl.pallas_call(
        functools.partial(_root_body, Cout),
        grid=grid,
        in_specs=[
            pl.BlockSpec((NC, BM, 128), lambda i: (0, i, 0)),
            pl.BlockSpec((BM, Cin), lambda i: (i, 0)),
            pl.BlockSpec((Cin, Cout), lambda i: (0, 0)),
            pl.BlockSpec((Cout,), lambda i: (0,)),
        ],
        out_specs=pl.BlockSpec((BM, Cout), lambda i: (i, 0)),
        out_shape=jax.ShapeDtypeStruct((N, Cout), jnp.float32),
    )(agg2, x, root, bias)


def _mlp_body(h_ref, w1_ref, b1_ref, w2_ref, b2_ref, o_ref):
    h = h_ref[...]
    t = _elu(jnp.dot(h, w1_ref[...], preferred_element_type=jnp.float32)
             + b1_ref[...][None, :])
    o_ref[...] = jnp.dot(t, w2_ref[...],
                         preferred_element_type=jnp.float32) + b2_ref[...][None, :]


def _mlp(h, w1, b1, w2, b2):
    BM = 2000
    grid = (N // BM,)
    return pl.pallas_call(
        _mlp_body,
        grid=grid,
        in_specs=[
            pl.BlockSpec((BM, h.shape[1]), lambda i: (i, 0)),
            pl.BlockSpec(w1.shape, lambda i: (0, 0)),
            pl.BlockSpec(b1.shape, lambda i: (0,)),
            pl.BlockSpec(w2.shape, lambda i: (0, 0)),
            pl.BlockSpec(b2.shape, lambda i: (0,)),
        ],
        out_specs=pl.BlockSpec((BM, w2.shape[1]), lambda i: (i, 0)),
        out_shape=jax.ShapeDtypeStruct((N, w2.shape[1]), jnp.float32),
    )(h, w1, b1, w2, b2)


def _spline_layer(h, gidx_f, goff_f, basis_f, dst, W, root, bias):
    K, Cin, Cout = W.shape
    wpad = jnp.pad(W, ((0, KP - K), (0, 0), (0, 0)))
    wflat = wpad.transpose(1, 0, 2).reshape(Cin, KP * Cout)
    xw = _xw_matmul(h, wflat)
    table = xw.reshape(xw.shape[0] * N, 128)
    agg2 = _make_sc_agg(Cout)(table, gidx_f, goff_f, basis_f, dst)
    return _root_combine(agg2, h, root, bias)


def kernel(x, edge_index, edge_attr, W1, root1, b1, W2, root2, b2, W3, root3, b3, W4, root4, b4, W5, root5, b5, W6, root6, b6, lin1_w, lin1_b, lin2_w, lin2_b):
    src, dst = edge_index[0], edge_index[1]
    basis2, g8, o8, g4, o4 = _prep(edge_attr, src)
    basis_f = basis2.reshape(E * S)
    g8f, o8f = g8.reshape(E * S), o8.reshape(E * S)
    g4f, o4f = g4.reshape(E * S), o4.reshape(E * S)
    h = x
    for i, (W, r, b) in enumerate([(W1, root1, b1), (W2, root2, b2),
                                   (W3, root3, b3), (W4, root4, b4),
                                   (W5, root5, b5), (W6, root6, b6)]):
        gf, of = (g8f, o8f) if W.shape[2] == 16 else (g4f, o4f)
        h = _spline_layer(h, gf, of, basis_f, dst, W, r, b)
    return _mlp(h, lin1_w, lin1_b, lin2_w, lin2_b)


# speedup vs baseline: 2.9637x; 1.4768x over previous
import functools

import jax
import jax.numpy as jnp
from jax import lax
from jax.experimental import pallas as pl
from jax.experimental.pallas import tpu as pltpu
from jax.experimental.pallas import tpu_sc as plsc

N = 10000
E = 160000
KS = 5
KTOT = KS ** 4
KP = 640
S = 16
NC = 2
NSUB = 16
NW = NC * NSUB
EPT = E // NW
B = 40
NB = EPT // B
NZT = 10
ZCH = 32


def _elu(v):
    return jnp.where(v > 0, v, jnp.exp(jnp.minimum(v, 0.0)) - 1.0)


def _prep_body(ea_ref, src_ref, dst_ref, basis_ref, g8_ref, o8_ref, g4_ref,
               o4_ref, dr8_ref, do8_ref, dr4_ref, do4_ref):
    ea = ea_ref[...]
    v = ea * (KS - 1.0)
    fl = jnp.floor(v)
    fr = v - fl
    fli = fl.astype(jnp.int32)
    src = src_ref[0, 0, :]
    w = None
    k = None
    for d in range(4):
        siota = lax.broadcasted_iota(jnp.int32, (1, S), 1)
        bits_d = (siota >> d) & 1
        bitf_d = bits_d.astype(jnp.float32)
        frd = fr[:, d:d + 1]
        wd = bitf_d * frd + (1.0 - bitf_d) * (1.0 - frd)
        idxd = jnp.clip(fli[:, d:d + 1] + bits_d, 0, KS - 1)
        w = wd if w is None else w * wd
        k = idxd * (KS ** d) if k is None else k + idxd * (KS ** d)
    srcb = src[:, None]
    dst = dst_ref[0, 0, :]
    dstb = dst[:, None]
    basis_ref[...] = w
    g8_ref[...] = (k // 8) * N + srcb
    o8_ref[...] = (k % 8) * 16
    g4_ref[...] = (k // 4) * N + srcb
    o4_ref[...] = (k % 4) * 32
    zs = jnp.zeros((1, S), jnp.int32)
    dr8_ref[0, 0, :] = dst // 8
    do8_ref[...] = (dstb % 8) * 16 + zs
    dr4_ref[0, 0, :] = dst // 4
    do4_ref[...] = (dstb % 4) * 32 + zs


def _prep(edge_attr, src, dst):
    BE = 4000
    grid = (E // BE,)
    src3 = src.reshape(E // BE, 1, BE)
    dst3 = dst.reshape(E // BE, 1, BE)
    espec = pl.BlockSpec((BE, S), lambda i: (i, 0))
    return pl.pallas_call(
        _prep_body,
        grid=grid,
        in_specs=[
            pl.BlockSpec((BE, 4), lambda i: (i, 0)),
            pl.BlockSpec((1, 1, BE), lambda i: (i, 0, 0)),
            pl.BlockSpec((1, 1, BE), lambda i: (i, 0, 0)),
        ],
        out_specs=[espec] * 5 + [
            pl.BlockSpec((1, 1, BE), lambda i: (i, 0, 0)),
            espec,
            pl.BlockSpec((1, 1, BE), lambda i: (i, 0, 0)),
            espec,
        ],
        out_shape=[jax.ShapeDtypeStruct((E, S), jnp.float32)]
        + [jax.ShapeDtypeStruct((E, S), jnp.int32)] * 4
        + [jax.ShapeDtypeStruct((E // BE, 1, BE), jnp.int32),
           jax.ShapeDtypeStruct((E, S), jnp.int32),
           jax.ShapeDtypeStruct((E // BE, 1, BE), jnp.int32),
           jax.ShapeDtypeStruct((E, S), jnp.int32)],
    )(edge_attr, src3, dst3)


def _xw_body(gb, x_ref, w_ref, o_ref):
    acc = jnp.dot(x_ref[...], w_ref[...], preferred_element_type=jnp.float32)
    for g in range(gb):
        o_ref[g] = acc[:, g * 128:(g + 1) * 128]


def _xw_matmul(x, wflat):
    M, Cin = x.shape
    KC = wflat.shape[1]
    GP = KC // 128
    BM, GB = 1000, 16
    grid = (M // BM, GP // GB)
    return pl.pallas_call(
        functools.partial(_xw_body, GB),
        grid=grid,
        in_specs=[
            pl.BlockSpec((BM, Cin), lambda i, j: (i, 0)),
            pl.BlockSpec((Cin, GB * 128), lambda i, j: (0, j)),
        ],
        out_specs=pl.BlockSpec((GB, BM, 128), lambda i, j: (j, i, 0)),
        out_shape=jax.ShapeDtypeStruct((GP, M, 128), jnp.float32),
    )(x, wflat)


@functools.cache
def _make_sc_agg(D):
    H = D // 16
    mesh = plsc.VectorSubcoreMesh(core_axis_name="c", subcore_axis_name="s")

    KPG = 128 // D
    NR = 2560 if D == 32 else 1280
    NPR = NR // NZT
    HB = B // 2

    @functools.partial(
        pl.kernel,
        out_type=jax.ShapeDtypeStruct((NC, NR, 128), jnp.float32),
        mesh=mesh,
        scratch_types=[
            pltpu.VMEM((ZCH, 128), jnp.float32),
            pltpu.VMEM_SHARED((NR, 128), jnp.float32),
        ],
    )
    def sc_agg(xw_hbm, gidx_hbm, goff_hbm, basis_hbm, dr_hbm, do_hbm, out_hbm,
               zb_v, agg_sh):
        cid = lax.axis_index("c")
        sid = lax.axis_index("s")
        wid = sid * NC + cid

        zrow = jnp.zeros((16,), jnp.float32)

        def zbody(i, carry):
            for h in range(8):
                zb_v[i, h * 16:(h + 1) * 16] = zrow
            return carry

        lax.fori_loop(0, ZCH, zbody, 0)

        @pl.when(sid < NZT)
        def _():
            for c in range(NPR // ZCH):
                pltpu.sync_copy(zb_v, agg_sh.at[pl.ds(sid * NPR + c * ZCH, ZCH)])

        plsc.subcore_barrier()

        ebase0 = wid * EPT

        @pl.loop(0, NB, unroll=1)
        def batch(ib):
            ebase = ebase0 + ib * B

            def inner(idx_v, off_v, bas_v, dof_v, dr_v, rows_v, msg_v, gsem):
                pltpu.sync_copy(goff_hbm.at[pl.ds(ebase * S, B * S)], off_v)
                pltpu.sync_copy(basis_hbm.at[pl.ds(ebase * S, B * S)], bas_v)
                pltpu.sync_copy(do_hbm.at[pl.ds(ebase * S, B * S)], dof_v)
                pltpu.sync_copy(dr_hbm.at[pl.ds(ebase, B)], dr_v)

                for half in range(2):
                    eoff = half * HB
                    pltpu.sync_copy(
                        gidx_hbm.at[pl.ds((ebase + eoff) * S, HB * S)], idx_v)
                    pltpu.async_copy(xw_hbm.at[idx_v], rows_v, gsem).wait()

                    @pl.loop(0, HB, unroll=1)
                    def ebody(j):
                        r0 = j * S
                        g0 = (eoff + j) * S
                        bvec = bas_v[pl.ds(g0, S)]
                        ovec = off_v[pl.ds(g0, S)]
                        dvec = dof_v[pl.ds(g0, S)]
                        doff = dvec[0]
                        acc = [jnp.zeros((16,), jnp.float32) for _ in range(H)]
                        for s in range(S):
                            b = bvec[s]
                            off = ovec[s]
                            for h in range(H):
                                acc[h] = acc[h] + rows_v[r0 + s, pl.ds(off + h * 16, 16)] * b
                        for h in range(8):
                            msg_v[eoff + j, h * 16:(h + 1) * 16] = zrow
                        for h in range(H):
                            msg_v[eoff + j, pl.ds(doff + h * 16, 16)] = acc[h]

                pltpu.sync_copy(msg_v, agg_sh.at[dr_v], add=True)

            pl.run_scoped(
                inner,
                pltpu.VMEM((HB * S,), jnp.int32),
                pltpu.VMEM((B * S,), jnp.int32),
                pltpu.VMEM((B * S,), jnp.float32),
                pltpu.VMEM((B * S,), jnp.int32),
                pltpu.VMEM((B,), jnp.int32),
                pltpu.VMEM((HB * S, 128), jnp.float32),
                pltpu.VMEM((B, 128), jnp.float32),
                pltpu.SemaphoreType.DMA,
            )

        plsc.subcore_barrier()

        @pl.when(sid < NZT)
        def _():
            for c in range(NPR // ZCH):
                pltpu.sync_copy(agg_sh.at[pl.ds(sid * NPR + c * ZCH, ZCH)], zb_v)
                pltpu.sync_copy(zb_v, out_hbm.at[cid, pl.ds(sid * NPR + c * ZCH, ZCH)])

    return sc_agg


def _root_body(agg_ref, x_ref, r_ref, b_ref, o_ref):
    a = agg_ref[0] + agg_ref[1]
    v = a + jnp.dot(x_ref[...], r_ref[...],
                    preferred_element_type=jnp.float32) + b_ref[...][None, :]
    o_ref[...] = _elu(v)


def _root_combine(agg2, x, root, bias):
    Cin, Cout = root.shape
    BM = 2000
    grid = (N // BM,)
    return pl.pallas_call(
        _root_body,
        grid=grid,
        in_specs=[
            pl.BlockSpec((NC, BM, Cout), lambda i: (0, i, 0)),
            pl.BlockSpec((BM, Cin), lambda i: (i, 0)),
            pl.BlockSpec((Cin, Cout), lambda i: (0, 0)),
            pl.BlockSpec((Cout,), lambda i: (0,)),
        ],
        out_specs=pl.BlockSpec((BM, Cout), lambda i: (i, 0)),
        out_shape=jax.ShapeDtypeStruct((N, Cout), jnp.float32),
    )(agg2, x, root, bias)


def _mlp_body(h_ref, w1_ref, b1_ref, w2_ref, b2_ref, o_ref):
    h = h_ref[...]
    t = _elu(jnp.dot(h, w1_ref[...], preferred_element_type=jnp.float32)
             + b1_ref[...][None, :])
    o_ref[...] = jnp.dot(t, w2_ref[...],
                         preferred_element_type=jnp.float32) + b2_ref[...][None, :]


def _mlp(h, w1, b1, w2, b2):
    BM = 2000
    grid = (N // BM,)
    return pl.pallas_call(
        _mlp_body,
        grid=grid,
        in_specs=[
            pl.BlockSpec((BM, h.shape[1]), lambda i: (i, 0)),
            pl.BlockSpec(w1.shape, lambda i: (0, 0)),
            pl.BlockSpec(b1.shape, lambda i: (0,)),
            pl.BlockSpec(w2.shape, lambda i: (0, 0)),
            pl.BlockSpec(b2.shape, lambda i: (0,)),
        ],
        out_specs=pl.BlockSpec((BM, w2.shape[1]), lambda i: (i, 0)),
        out_shape=jax.ShapeDtypeStruct((N, w2.shape[1]), jnp.float32),
    )(h, w1, b1, w2, b2)


def _spline_layer(h, gidx_f, goff_f, basis_f, dr_f, do_f, W, root, bias):
    K, Cin, Cout = W.shape
    wpad = jnp.pad(W, ((0, KP - K), (0, 0), (0, 0)))
    wflat = wpad.transpose(1, 0, 2).reshape(Cin, KP * Cout)
    xw = _xw_matmul(h, wflat)
    table = xw.reshape(xw.shape[0] * N, 128)
    agg2 = _make_sc_agg(Cout)(table, gidx_f, goff_f, basis_f, dr_f, do_f)
    kpg = 128 // Cout
    aggl = agg2.reshape(NC, agg2.shape[1] * kpg, Cout)[:, :N, :]
    return _root_combine(aggl, h, root, bias)


def kernel(x, edge_index, edge_attr, W1, root1, b1, W2, root2, b2, W3, root3, b3, W4, root4, b4, W5, root5, b5, W6, root6, b6, lin1_w, lin1_b, lin2_w, lin2_b):
    src, dst = edge_index[0], edge_index[1]
    basis2, g8, o8, g4, o4, dr8, do8, dr4, do4 = _prep(edge_attr, src, dst)
    basis_f = basis2.reshape(E * S)
    g8f, o8f = g8.reshape(E * S), o8.reshape(E * S)
    g4f, o4f = g4.reshape(E * S), o4.reshape(E * S)
    dr8f, do8f = dr8.reshape(E), do8.reshape(E * S)
    dr4f, do4f = dr4.reshape(E), do4.reshape(E * S)
    h = x
    for i, (W, r, b) in enumerate([(W1, root1, b1), (W2, root2, b2),
                                   (W3, root3, b3), (W4, root4, b4),
                                   (W5, root5, b5), (W6, root6, b6)]):
        if W.shape[2] == 16:
            gf, of, drf, dof = g8f, o8f, dr8f, do8f
        else:
            gf, of, drf, dof = g4f, o4f, dr4f, do4f
        h = _spline_layer(h, gf, of, basis_f, drf, dof, W, r, b)
    return _mlp(h, lin1_w, lin1_b, lin2_w, lin2_b)


# speedup vs baseline: 3.2453x; 1.0950x over previous
import functools

import jax
import jax.numpy as jnp
from jax import lax
from jax.experimental import pallas as pl
from jax.experimental.pallas import tpu as pltpu
from jax.experimental.pallas import tpu_sc as plsc

N = 10000
E = 160000
KS = 5
KTOT = KS ** 4
KP = 640
S = 16
NC = 2
NSUB = 16
NW = NC * NSUB
EPT = E // NW
B = 40
NB = EPT // B
NZT = 10
ZCH = 32


def _elu(v):
    return jnp.where(v > 0, v, jnp.exp(jnp.minimum(v, 0.0)) - 1.0)


def _prep_body(ea_ref, src_ref, dst_ref, basis_ref, g8_ref, o8_ref, g4_ref,
               o4_ref, dr8_ref, do8_ref, dr4_ref, do4_ref):
    ea = ea_ref[...]
    v = ea * (KS - 1.0)
    fl = jnp.floor(v)
    fr = v - fl
    fli = fl.astype(jnp.int32)
    src = src_ref[0, 0, :]
    w = None
    k = None
    for d in range(4):
        siota = lax.broadcasted_iota(jnp.int32, (1, S), 1)
        bits_d = (siota >> d) & 1
        bitf_d = bits_d.astype(jnp.float32)
        frd = fr[:, d:d + 1]
        wd = bitf_d * frd + (1.0 - bitf_d) * (1.0 - frd)
        idxd = jnp.clip(fli[:, d:d + 1] + bits_d, 0, KS - 1)
        w = wd if w is None else w * wd
        k = idxd * (KS ** d) if k is None else k + idxd * (KS ** d)
    srcb = src[:, None]
    dst = dst_ref[0, 0, :]
    dstb = dst[:, None]
    basis_ref[...] = w
    g8_ref[...] = (k // 8) * N + srcb
    o8_ref[...] = (k % 8) * 16
    g4_ref[...] = (k // 4) * N + srcb
    o4_ref[...] = (k % 4) * 32
    zs = jnp.zeros((1, S), jnp.int32)
    dr8_ref[0, 0, :] = dst // 8
    do8_ref[...] = (dstb % 8) * 16 + zs
    dr4_ref[0, 0, :] = dst // 4
    do4_ref[...] = (dstb % 4) * 32 + zs


def _prep(edge_attr, src, dst):
    BE = 4000
    grid = (E // BE,)
    src3 = src.reshape(E // BE, 1, BE)
    dst3 = dst.reshape(E // BE, 1, BE)
    espec = pl.BlockSpec((BE, S), lambda i: (i, 0))
    return pl.pallas_call(
        _prep_body,
        grid=grid,
        in_specs=[
            pl.BlockSpec((BE, 4), lambda i: (i, 0)),
            pl.BlockSpec((1, 1, BE), lambda i: (i, 0, 0)),
            pl.BlockSpec((1, 1, BE), lambda i: (i, 0, 0)),
        ],
        out_specs=[espec] * 5 + [
            pl.BlockSpec((1, 1, BE), lambda i: (i, 0, 0)),
            espec,
            pl.BlockSpec((1, 1, BE), lambda i: (i, 0, 0)),
            espec,
        ],
        out_shape=[jax.ShapeDtypeStruct((E, S), jnp.float32)]
        + [jax.ShapeDtypeStruct((E, S), jnp.int32)] * 4
        + [jax.ShapeDtypeStruct((E // BE, 1, BE), jnp.int32),
           jax.ShapeDtypeStruct((E, S), jnp.int32),
           jax.ShapeDtypeStruct((E // BE, 1, BE), jnp.int32),
           jax.ShapeDtypeStruct((E, S), jnp.int32)],
    )(edge_attr, src3, dst3)


def _xw_body(gb, x_ref, w_ref, o_ref):
    acc = jnp.dot(x_ref[...], w_ref[...], preferred_element_type=jnp.float32)
    for g in range(gb):
        o_ref[g] = acc[:, g * 128:(g + 1) * 128]


def _xw_matmul(x, wflat):
    M, Cin = x.shape
    KC = wflat.shape[1]
    GP = KC // 128
    BM, GB = 1000, 16
    grid = (M // BM, GP // GB)
    return pl.pallas_call(
        functools.partial(_xw_body, GB),
        grid=grid,
        in_specs=[
            pl.BlockSpec((BM, Cin), lambda i, j: (i, 0)),
            pl.BlockSpec((Cin, GB * 128), lambda i, j: (0, j)),
        ],
        out_specs=pl.BlockSpec((GB, BM, 128), lambda i, j: (j, i, 0)),
        out_shape=jax.ShapeDtypeStruct((GP, M, 128), jnp.float32),
    )(x, wflat)


@functools.cache
def _make_sc_agg(D):
    H = D // 16
    mesh = plsc.VectorSubcoreMesh(core_axis_name="c", subcore_axis_name="s")

    KPG = 128 // D
    NR = 2560 if D == 32 else 1280
    NPR = NR // NZT
    QB = B // 4

    @functools.partial(
        pl.kernel,
        out_type=jax.ShapeDtypeStruct((NC, NR, 128), jnp.float32),
        mesh=mesh,
        scratch_types=[
            pltpu.VMEM((ZCH, 128), jnp.float32),
            pltpu.VMEM_SHARED((NR, 128), jnp.float32),
        ],
    )
    def sc_agg(xw_hbm, gidx_hbm, goff_hbm, basis_hbm, dr_hbm, do_hbm, out_hbm,
               zb_v, agg_sh):
        cid = lax.axis_index("c")
        sid = lax.axis_index("s")
        wid = sid * NC + cid

        zrow = jnp.zeros((16,), jnp.float32)

        def zbody(i, carry):
            for h in range(8):
                zb_v[i, h * 16:(h + 1) * 16] = zrow
            return carry

        lax.fori_loop(0, ZCH, zbody, 0)

        @pl.when(sid < NZT)
        def _():
            for c in range(NPR // ZCH):
                pltpu.sync_copy(zb_v, agg_sh.at[pl.ds(sid * NPR + c * ZCH, ZCH)])

        plsc.subcore_barrier()

        ebase0 = wid * EPT

        @pl.loop(0, NB, unroll=1)
        def batch(ib):
            ebase = ebase0 + ib * B

            def inner(idx0_v, idx1_v, off_v, bas_v, dof_v, dr_v,
                      rows0_v, rows1_v, msg_v, sem0, sem1):
                idxs = [idx0_v, idx1_v]
                rows = [rows0_v, rows1_v]
                sems = [sem0, sem1]
                pltpu.sync_copy(goff_hbm.at[pl.ds(ebase * S, B * S)], off_v)
                pltpu.sync_copy(basis_hbm.at[pl.ds(ebase * S, B * S)], bas_v)
                pltpu.sync_copy(do_hbm.at[pl.ds(ebase * S, B * S)], dof_v)
                pltpu.sync_copy(dr_hbm.at[pl.ds(ebase, B)], dr_v)

                pltpu.sync_copy(gidx_hbm.at[pl.ds(ebase * S, QB * S)], idx0_v)
                descs = [pltpu.async_copy(xw_hbm.at[idx0_v], rows0_v, sem0)]

                for q in range(4):
                    if q < 3:
                        nb = (q + 1) % 2
                        pltpu.sync_copy(
                            gidx_hbm.at[pl.ds((ebase + (q + 1) * QB) * S, QB * S)],
                            idxs[nb])
                        descs.append(
                            pltpu.async_copy(xw_hbm.at[idxs[nb]], rows[nb], sems[nb]))
                    descs[q].wait()
                    eoff = q * QB
                    rq = rows[q % 2]

                    @pl.loop(0, QB, unroll=1)
                    def ebody(j):
                        r0 = j * S
                        g0 = (eoff + j) * S
                        bvec = bas_v[pl.ds(g0, S)]
                        ovec = off_v[pl.ds(g0, S)]
                        dvec = dof_v[pl.ds(g0, S)]
                        doff = dvec[0]
                        acc = [jnp.zeros((16,), jnp.float32) for _ in range(H)]
                        for s in range(S):
                            b = bvec[s]
                            off = ovec[s]
                            for h in range(H):
                                acc[h] = acc[h] + rq[r0 + s, pl.ds(off + h * 16, 16)] * b
                        for h in range(8):
                            msg_v[eoff + j, h * 16:(h + 1) * 16] = zrow
                        for h in range(H):
                            msg_v[eoff + j, pl.ds(doff + h * 16, 16)] = acc[h]

                pltpu.sync_copy(msg_v, agg_sh.at[dr_v], add=True)

            pl.run_scoped(
                inner,
                pltpu.VMEM((QB * S,), jnp.int32),
                pltpu.VMEM((QB * S,), jnp.int32),
                pltpu.VMEM((B * S,), jnp.int32),
                pltpu.VMEM((B * S,), jnp.float32),
                pltpu.VMEM((B * S,), jnp.int32),
                pltpu.VMEM((B,), jnp.int32),
                pltpu.VMEM((QB * S, 128), jnp.float32),
                pltpu.VMEM((QB * S, 128), jnp.float32),
                pltpu.VMEM((B, 128), jnp.float32),
                pltpu.SemaphoreType.DMA,
                pltpu.SemaphoreType.DMA,
            )

        plsc.subcore_barrier()

        @pl.when(sid < NZT)
        def _():
            for c in range(NPR // ZCH):
                pltpu.sync_copy(agg_sh.at[pl.ds(sid * NPR + c * ZCH, ZCH)], zb_v)
                pltpu.sync_copy(zb_v, out_hbm.at[cid, pl.ds(sid * NPR + c * ZCH, ZCH)])

    return sc_agg


def _root_body(agg_ref, x_ref, r_ref, b_ref, o_ref):
    a = agg_ref[0] + agg_ref[1]
    v = a + jnp.dot(x_ref[...], r_ref[...],
                    preferred_element_type=jnp.float32) + b_ref[...][None, :]
    o_ref[...] = _elu(v)


def _root_combine(agg2, x, root, bias):
    Cin, Cout = root.shape
    BM = 2000
    grid = (N // BM,)
    return pl.pallas_call(
        _root_body,
        grid=grid,
        in_specs=[
            pl.BlockSpec((NC, BM, Cout), lambda i: (0, i, 0)),
            pl.BlockSpec((BM, Cin), lambda i: (i, 0)),
            pl.BlockSpec((Cin, Cout), lambda i: (0, 0)),
            pl.BlockSpec((Cout,), lambda i: (0,)),
        ],
        out_specs=pl.BlockSpec((BM, Cout), lambda i: (i, 0)),
        out_shape=jax.ShapeDtypeStruct((N, Cout), jnp.float32),
    )(agg2, x, root, bias)


def _mlp_body(h_ref, w1_ref, b1_ref, w2_ref, b2_ref, o_ref):
    h = h_ref[...]
    t = _elu(jnp.dot(h, w1_ref[...], preferred_element_type=jnp.float32)
             + b1_ref[...][None, :])
    o_ref[...] = jnp.dot(t, w2_ref[...],
                         preferred_element_type=jnp.float32) + b2_ref[...][None, :]


def _mlp(h, w1, b1, w2, b2):
    BM = 2000
    grid = (N // BM,)
    return pl.pallas_call(
        _mlp_body,
        grid=grid,
        in_specs=[
            pl.BlockSpec((BM, h.shape[1]), lambda i: (i, 0)),
            pl.BlockSpec(w1.shape, lambda i: (0, 0)),
            pl.BlockSpec(b1.shape, lambda i: (0,)),
            pl.BlockSpec(w2.shape, lambda i: (0, 0)),
            pl.BlockSpec(b2.shape, lambda i: (0,)),
        ],
        out_specs=pl.BlockSpec((BM, w2.shape[1]), lambda i: (i, 0)),
        out_shape=jax.ShapeDtypeStruct((N, w2.shape[1]), jnp.float32),
    )(h, w1, b1, w2, b2)


def _spline_layer(h, gidx_f, goff_f, basis_f, dr_f, do_f, W, root, bias):
    K, Cin, Cout = W.shape
    wpad = jnp.pad(W, ((0, KP - K), (0, 0), (0, 0)))
    wflat = wpad.transpose(1, 0, 2).reshape(Cin, KP * Cout)
    xw = _xw_matmul(h, wflat)
    table = xw.reshape(xw.shape[0] * N, 128)
    agg2 = _make_sc_agg(Cout)(table, gidx_f, goff_f, basis_f, dr_f, do_f)
    kpg = 128 // Cout
    aggl = agg2.reshape(NC, agg2.shape[1] * kpg, Cout)[:, :N, :]
    return _root_combine(aggl, h, root, bias)


def kernel(x, edge_index, edge_attr, W1, root1, b1, W2, root2, b2, W3, root3, b3, W4, root4, b4, W5, root5, b5, W6, root6, b6, lin1_w, lin1_b, lin2_w, lin2_b):
    src, dst = edge_index[0], edge_index[1]
    basis2, g8, o8, g4, o4, dr8, do8, dr4, do4 = _prep(edge_attr, src, dst)
    basis_f = basis2.reshape(E * S)
    g8f, o8f = g8.reshape(E * S), o8.reshape(E * S)
    g4f, o4f = g4.reshape(E * S), o4.reshape(E * S)
    dr8f, do8f = dr8.reshape(E), do8.reshape(E * S)
    dr4f, do4f = dr4.reshape(E), do4.reshape(E * S)
    h = x
    for i, (W, r, b) in enumerate([(W1, root1, b1), (W2, root2, b2),
                                   (W3, root3, b3), (W4, root4, b4),
                                   (W5, root5, b5), (W6, root6, b6)]):
        if W.shape[2] == 16:
            gf, of, drf, dof = g8f, o8f, dr8f, do8f
        else:
            gf, of, drf, dof = g4f, o4f, dr4f, do4f
        h = _spline_layer(h, gf, of, basis_f, drf, dof, W, r, b)
    return _mlp(h, lin1_w, lin1_b, lin2_w, lin2_b)
